# baseline (speedup 1.0000x reference)
import functools

import jax
import jax.numpy as jnp
from jax.experimental import pallas as pl

B = 8
NUM_SLOTS = 8192
SLOT_DIM = 64
N_HEADS = 8
TOPK = 16
BOTTLENECK = 64

_SQRT2 = 1.4142135623730951


def _body(mem_ref, keys_ref, vals_ref, erase_ref, addg_ref, beta_ref,
          w1_ref, b1_ref, w2_ref, b2_ref, decay_ref, age_ref,
          newmem_ref, w_out_ref):
    b = pl.program_id(0)
    mem = mem_ref[0]
    keys = keys_ref[b]
    vals = vals_ref[b]
    erase = erase_ref[b]
    addg = addg_ref[b]
    beta = beta_ref[b]

    h = jax.lax.dot_general(vals, w1_ref[...], (((1,), (0,)), ((), ())),
                            preferred_element_type=jnp.float32) + b1_ref[...]
    h = 0.5 * h * (1.0 + jax.lax.erf(h / _SQRT2))
    cv = jax.lax.dot_general(h, w2_ref[...], (((1,), (0,)), ((), ())),
                             preferred_element_type=jnp.float32) + b2_ref[...]
    cvg = cv * addg

    kn = keys / jnp.maximum(
        jnp.sqrt(jnp.sum(keys * keys, axis=1, keepdims=True)), 1e-12)
    mem_n = mem / jnp.maximum(
        jnp.sqrt(jnp.sum(mem * mem, axis=1, keepdims=True)), 1e-12)

    sim = jax.lax.dot_general(kn, mem_n, (((1,), (1,)), ((), ())),
                              preferred_element_type=jnp.float32)
    a1 = age_ref[...] + 1.0
    age_bias = a1 / (jnp.max(a1) + 1e-8)
    sim = sim * beta + age_bias

    iota = jax.lax.broadcasted_iota(jnp.int32, (N_HEADS, NUM_SLOTS), 1)
    work = sim
    mask = jnp.zeros((N_HEADS, NUM_SLOTS), dtype=jnp.bool_)
    m0 = jnp.max(work, axis=1, keepdims=True)
    for _ in range(TOPK):
        m = jnp.max(work, axis=1, keepdims=True)
        cand = jnp.where(work == m, iota, NUM_SLOTS)
        first = jnp.min(cand, axis=1, keepdims=True)
        sel = iota == first
        mask = jnp.logical_or(mask, sel)
        work = jnp.where(sel, -jnp.inf, work)

    wexp = jnp.where(mask, jnp.exp(sim - m0), 0.0)
    w = wexp / jnp.sum(wexp, axis=1, keepdims=True)
    w_out_ref[0] = w

    scale = 1.0 / N_HEADS
    e_col = jax.lax.dot_general(w, erase * scale, (((0,), (0,)), ((), ())),
                                preferred_element_type=jnp.float32)
    a_mat = jax.lax.dot_general(w, cvg * scale, (((0,), (0,)), ((), ())),
                                preferred_element_type=jnp.float32)
    new = mem * (1.0 - e_col) + a_mat
    new = new + 1e-8
    nn = jnp.maximum(
        jnp.sqrt(jnp.sum(new * new, axis=1, keepdims=True)), 1e-12)
    dec = jax.nn.sigmoid(decay_ref[...])
    newmem_ref[0] = new / nn * dec


@jax.jit
def kernel(memory, write_keys, write_vals, erase, add_gate, beta,
           W1, b1, W2, b2, decay_gate, age):
    full = lambda s: pl.BlockSpec(s, lambda b: tuple(0 for _ in s))
    grid_spec = pl.GridSpec(
        grid=(B,),
        in_specs=[
            pl.BlockSpec((1, NUM_SLOTS, SLOT_DIM), lambda b: (b, 0, 0)),
            full((B, N_HEADS, SLOT_DIM)),
            full((B, N_HEADS, SLOT_DIM)),
            full((B, N_HEADS, 1)),
            full((B, N_HEADS, 1)),
            full((B, N_HEADS, 1)),
            full((SLOT_DIM, BOTTLENECK)),
            full((1, BOTTLENECK)),
            full((BOTTLENECK, SLOT_DIM)),
            full((1, SLOT_DIM)),
            full((NUM_SLOTS, 1)),
            full((1, NUM_SLOTS)),
        ],
        out_specs=[
            pl.BlockSpec((1, NUM_SLOTS, SLOT_DIM), lambda b: (b, 0, 0)),
            pl.BlockSpec((1, N_HEADS, NUM_SLOTS), lambda b: (b, 0, 0)),
        ],
    )
    new_memory, weights = pl.pallas_call(
        _body,
        grid_spec=grid_spec,
        out_shape=[
            jax.ShapeDtypeStruct((B, NUM_SLOTS, SLOT_DIM), jnp.float32),
            jax.ShapeDtypeStruct((B, N_HEADS, NUM_SLOTS), jnp.float32),
        ],
    )(memory, write_keys, write_vals,
      erase[..., None], add_gate[..., None], beta[..., None],
      W1, b1.reshape(1, BOTTLENECK), W2, b2.reshape(1, SLOT_DIM),
      decay_gate.reshape(NUM_SLOTS, 1), age)
    return (new_memory, weights)


# speedup vs baseline: 1.5526x; 1.5526x over previous
import functools

import jax
import jax.numpy as jnp
from jax.experimental import pallas as pl

B = 8
NUM_SLOTS = 8192
SLOT_DIM = 64
N_HEADS = 8
TOPK = 16
BOTTLENECK = 64

_SQRT2 = 1.4142135623730951


def _body(mem_ref, keys_ref, vals_ref, erase_ref, addg_ref, beta_ref,
          w1_ref, b1_ref, w2_ref, b2_ref, decay_ref, age_ref,
          newmem_ref, w_out_ref):
    b = pl.program_id(0)
    mem = mem_ref[0]
    keys = keys_ref[b]
    vals = vals_ref[b]
    erase = erase_ref[b]
    addg = addg_ref[b]
    beta = beta_ref[b]

    h = jax.lax.dot_general(vals, w1_ref[...], (((1,), (0,)), ((), ())),
                            preferred_element_type=jnp.float32) + b1_ref[...]
    h = 0.5 * h * (1.0 + jax.lax.erf(h / _SQRT2))
    cv = jax.lax.dot_general(h, w2_ref[...], (((1,), (0,)), ((), ())),
                             preferred_element_type=jnp.float32) + b2_ref[...]
    cvg = cv * addg

    kn = keys / jnp.maximum(
        jnp.sqrt(jnp.sum(keys * keys, axis=1, keepdims=True)), 1e-12)
    ones_row = jnp.ones((1, SLOT_DIM), jnp.float32)
    s_mem = jax.lax.dot_general(ones_row, mem * mem, (((1,), (1,)), ((), ())),
                                preferred_element_type=jnp.float32)
    inv_mem = 1.0 / jnp.maximum(jnp.sqrt(s_mem), 1e-12)

    sim = jax.lax.dot_general(kn, mem, (((1,), (1,)), ((), ())),
                              preferred_element_type=jnp.float32)
    a1 = age_ref[...] + 1.0
    age_bias = a1 / (jnp.max(a1) + 1e-8)
    sim = sim * inv_mem * beta + age_bias

    iota = jax.lax.broadcasted_iota(jnp.int32, (N_HEADS, NUM_SLOTS), 1)
    work = sim
    mask = jnp.zeros((N_HEADS, NUM_SLOTS), dtype=jnp.bool_)
    m0 = jnp.max(work, axis=1, keepdims=True)
    for _ in range(TOPK):
        m = jnp.max(work, axis=1, keepdims=True)
        cand = jnp.where(work == m, iota, NUM_SLOTS)
        first = jnp.min(cand, axis=1, keepdims=True)
        sel = iota == first
        mask = jnp.logical_or(mask, sel)
        work = jnp.where(sel, -jnp.inf, work)

    wexp = jnp.where(mask, jnp.exp(sim - m0), 0.0)
    w = wexp / jnp.sum(wexp, axis=1, keepdims=True)
    w_out_ref[0] = w

    scale = 1.0 / N_HEADS
    e_col = jax.lax.dot_general(w, erase * scale, (((0,), (0,)), ((), ())),
                                preferred_element_type=jnp.float32)
    a_mat = jax.lax.dot_general(w, cvg * scale, (((0,), (0,)), ((), ())),
                                preferred_element_type=jnp.float32)
    new = mem * (1.0 - e_col) + a_mat + 1e-8
    ones_col = jnp.ones((SLOT_DIM, 1), jnp.float32)
    s_new = jax.lax.dot_general(new * new, ones_col, (((1,), (0,)), ((), ())),
                                preferred_element_type=jnp.float32)
    dec = jax.nn.sigmoid(decay_ref[...])
    newmem_ref[0] = new * (dec / jnp.maximum(jnp.sqrt(s_new), 1e-12))


@jax.jit
def kernel(memory, write_keys, write_vals, erase, add_gate, beta,
           W1, b1, W2, b2, decay_gate, age):
    full = lambda s: pl.BlockSpec(s, lambda b: tuple(0 for _ in s))
    grid_spec = pl.GridSpec(
        grid=(B,),
        in_specs=[
            pl.BlockSpec((1, NUM_SLOTS, SLOT_DIM), lambda b: (b, 0, 0)),
            full((B, N_HEADS, SLOT_DIM)),
            full((B, N_HEADS, SLOT_DIM)),
            full((B, N_HEADS, 1)),
            full((B, N_HEADS, 1)),
            full((B, N_HEADS, 1)),
            full((SLOT_DIM, BOTTLENECK)),
            full((1, BOTTLENECK)),
            full((BOTTLENECK, SLOT_DIM)),
            full((1, SLOT_DIM)),
            full((NUM_SLOTS, 1)),
            full((1, NUM_SLOTS)),
        ],
        out_specs=[
            pl.BlockSpec((1, NUM_SLOTS, SLOT_DIM), lambda b: (b, 0, 0)),
            pl.BlockSpec((1, N_HEADS, NUM_SLOTS), lambda b: (b, 0, 0)),
        ],
    )
    new_memory, weights = pl.pallas_call(
        _body,
        grid_spec=grid_spec,
        out_shape=[
            jax.ShapeDtypeStruct((B, NUM_SLOTS, SLOT_DIM), jnp.float32),
            jax.ShapeDtypeStruct((B, N_HEADS, NUM_SLOTS), jnp.float32),
        ],
    )(memory, write_keys, write_vals,
      erase[..., None], add_gate[..., None], beta[..., None],
      W1, b1.reshape(1, BOTTLENECK), W2, b2.reshape(1, SLOT_DIM),
      decay_gate.reshape(NUM_SLOTS, 1), age)
    return (new_memory, weights)
